# baseline (speedup 1.0000x reference)
import jax, jax.numpy as jnp
from jax import lax
from jax.experimental import pallas as pl

_NC = 64
_C = 12800


def _body(idx_ref, a_ref, b_ref):
    idx = idx_ref[0]
    iota = lax.broadcasted_iota(jnp.int32, (_NC, _C), 0)
    oh = (iota == idx).astype(jnp.float32)
    a_ref[...] = oh
    b_ref[...] = oh


def kernel(species_index, pos):
    n = species_index.shape[0]
    g = (n + _C - 1) // _C
    idx_p = jnp.pad(species_index.astype(jnp.int32), (0, g * _C - n))
    idx3 = idx_p.reshape(g, 1, _C)
    spec = pl.BlockSpec((_NC, _C), lambda i: (0, i))
    a, b = pl.pallas_call(
        _body,
        grid=(g,),
        in_specs=[pl.BlockSpec((1, 1, _C), lambda i: (i, 0, 0))],
        out_specs=[spec, spec],
        out_shape=[jax.ShapeDtypeStruct((_NC, n), jnp.float32)] * 2,
    )(idx3)
    return a.T, b.T
